# baseline (speedup 1.0000x reference)
import functools

import jax
import jax.numpy as jnp
from jax import lax
from jax.experimental import pallas as pl
from jax.experimental.pallas import tpu as pltpu
from jax.experimental.pallas import tpu_sc as plsc

N = 10000
NPAD = 10240
W = 88
NSUB = 16
NCORE = 2
NWIN = 235
EDGES_PER_SUB = NWIN * W
EP = NSUB * EDGES_PER_SUB
RB = 10
BR = N // RB
ZROWS = 640
OROWS = 400

IB = 8
GB = 4
SB = 4
UNROLL = 8
PFD = 4
LEAD = 1
LAG = 3


@functools.lru_cache(maxsize=None)
def _make_spmm(nfb, extract_scale):
  fpc = nfb // NCORE
  mesh = plsc.VectorSubcoreMesh(core_axis_name="c", subcore_axis_name="s",
                                num_cores=NCORE, num_subcores=NSUB)

  out_type = [jax.ShapeDtypeStruct((nfb, N, 128), jnp.float32)]
  if extract_scale:
    out_type.append(jax.ShapeDtypeStruct((N, 128), jnp.float32))

  scratch = (
      [pltpu.VMEM((2, W), jnp.int32) for _ in range(IB)] +
      [pltpu.VMEM((W, 128), jnp.float32) for _ in range(GB)] +
      [pltpu.VMEM_SHARED((NPAD, 128), jnp.float32)] +
      [pltpu.SemaphoreType.DMA for _ in range(IB + GB + SB)]
  )

  def body(hw, idxr, zerosr, *rest):
    if extract_scale:
      onesr, segr, cntr = rest[:3]
      rest = rest[3:]
    else:
      segr = rest[0]
      rest = rest[1:]
    idx_v = rest[:IB]
    g_v = rest[IB:IB + GB]
    acc_sh = rest[IB + GB]
    sem_i = rest[IB + GB + 1:IB + GB + 1 + IB]
    sem_g = rest[IB + GB + 1 + IB:IB + GB + 1 + IB + GB]
    sem_s = rest[IB + GB + 1 + IB + GB:]
    c = lax.axis_index("c")
    s = lax.axis_index("s")

    def idx_issue(w, m):
      pltpu.async_copy(idxr.at[s * NWIN + w], idx_v[m], sem_i[m])

    def idx_wait(w, m):
      pltpu.make_async_copy(idxr.at[s * NWIN + w], idx_v[m], sem_i[m]).wait()

    def writeout(dst):
      pltpu.sync_copy(acc_sh.at[pl.ds(s * OROWS, OROWS)],
                      dst.at[pl.ds(s * OROWS, OROWS)])

      @pl.when(s < 9)
      def _():
        pltpu.sync_copy(acc_sh.at[pl.ds((s + 16) * OROWS, OROWS)],
                        dst.at[pl.ds((s + 16) * OROWS, OROWS)])

    def run_pass(sc_issue, sc_wait, gather_issue, gather_wait, dst):
      pltpu.sync_copy(zerosr, acc_sh.at[pl.ds(s * ZROWS, ZROWS)])
      plsc.subcore_barrier()

      def bodyw(w, m, skip_scwait=False, do_idx=True, do_next=True):
        if not skip_scwait:
          sc_wait(w - LAG, (m - LAG) % IB, (m - LAG) % SB, (m - LAG) % GB)
        if do_idx:
          idx_issue(w + PFD, (m + PFD) % IB)
        if do_next:
          idx_wait(w + LEAD, (m + LEAD) % IB)
          gather_issue(w + LEAD, (m + LEAD) % IB, (m + LEAD) % GB)
        gather_wait(w, m % IB, m % GB)
        sc_issue(w, m % IB, m % SB, m % GB)

      for w in range(PFD):
        idx_issue(w, w)
      for w in range(LEAD):
        idx_wait(w, w)
        gather_issue(w, w, w)
      for w in range(LAG):
        bodyw(w, w, skip_scwait=True)

      k_iters = (NWIN - LAG - PFD) // UNROLL
      tail_start = LAG + UNROLL * k_iters

      @pl.loop(LAG, tail_start, step=UNROLL)
      def _(t):
        for k in range(UNROLL):
          bodyw(t + k, (LAG + k) % UNROLL)

      for w in range(tail_start, NWIN):
        bodyw(w, w % UNROLL, do_idx=(w + PFD < NWIN),
              do_next=(w + LEAD < NWIN))
      for w in range(NWIN - LAG, NWIN):
        sc_wait(w, w % IB, w % SB, w % GB)

      plsc.subcore_barrier()
      writeout(dst)
      plsc.subcore_barrier()

    def mk_gather(fb):
      def gather_issue(w, m8, m4):
        pltpu.async_copy(hw.at[fb].at[idx_v[m8].at[1]], g_v[m4], sem_g[m4])

      def gather_wait(w, m8, m4):
        pltpu.make_async_copy(hw.at[fb].at[idx_v[m8].at[1]], g_v[m4],
                              sem_g[m4]).wait()

      def sc_issue(w, m8, msem, m4):
        pltpu.async_copy(g_v[m4], acc_sh.at[idx_v[m8].at[0]], sem_s[msem],
                         add=True)

      def sc_wait(w, m8, msem, m4):
        pltpu.make_async_copy(g_v[m4], acc_sh.at[idx_v[m8].at[0]],
                              sem_s[msem]).wait()

      return gather_issue, gather_wait, sc_issue, sc_wait

    if extract_scale:
      @pl.when(c == 0)
      def _():
        pltpu.sync_copy(onesr, g_v[0])

        def gather_issue(w, m8, m4):
          pass

        def gather_wait(w, m8, m4):
          pass

        def sc_issue(w, m8, msem, m4):
          pltpu.async_copy(g_v[0], acc_sh.at[idx_v[m8].at[0]], sem_s[msem],
                           add=True)

        def sc_wait(w, m8, msem, m4):
          pltpu.make_async_copy(g_v[0], acc_sh.at[idx_v[m8].at[0]],
                                sem_s[msem]).wait()

        run_pass(sc_issue, sc_wait, gather_issue, gather_wait, cntr)

    for j in range(fpc):
      fb = c * fpc + j
      gi, gw, si, sw = mk_gather(fb)
      run_pass(si, sw, gi, gw, segr.at[fb])

  return pl.kernel(body, out_type=tuple(out_type), mesh=mesh,
                   scratch_types=scratch)


def _spmm_first(*args):
  return _make_spmm(4, True)(*args)


def _spmm_mid(*args):
  return _make_spmm(4, False)(*args)


def _spmm_last(*args):
  return _make_spmm(2, False)(*args)


def _mm0_body(x_ref, w_ref, o_ref):
  o_ref[0] = jnp.dot(x_ref[...], w_ref[...],
                     preferred_element_type=jnp.float32)


def _mm0(x, w):
  return pl.pallas_call(
      _mm0_body,
      grid=(RB, 4),
      in_specs=[
          pl.BlockSpec((BR, 256), lambda r, n: (r, 0)),
          pl.BlockSpec((256, 128), lambda r, n: (0, n)),
      ],
      out_specs=pl.BlockSpec((1, BR, 128), lambda r, n: (n, r, 0)),
      out_shape=jax.ShapeDtypeStruct((4, N, 128), jnp.float32),
      compiler_params=pltpu.CompilerParams(
          dimension_semantics=("parallel", "parallel")),
  )(x, w)


def _mid_body(seg_ref, scl_ref, b_ref, w_ref, o_ref):
  k = pl.program_id(2)
  t = seg_ref[0] * (1.0 / scl_ref[:, 0:1]) + b_ref[0, 0]
  t = jnp.where(t >= 0, t, 0.2 * t)
  p = jnp.dot(t, w_ref[...], preferred_element_type=jnp.float32)

  @pl.when(k == 0)
  def _():
    o_ref[0] = p

  @pl.when(k > 0)
  def _():
    o_ref[0] += p


def _mid(seg, scl, b, w, nfb_out):
  nfb_in = seg.shape[0]
  return pl.pallas_call(
      _mid_body,
      grid=(RB, nfb_out, nfb_in),
      in_specs=[
          pl.BlockSpec((1, BR, 128), lambda r, n, k: (k, r, 0)),
          pl.BlockSpec((BR, 128), lambda r, n, k: (r, 0)),
          pl.BlockSpec((1, 1, 128), lambda r, n, k: (k, 0, 0)),
          pl.BlockSpec((128, 128), lambda r, n, k: (k, n)),
      ],
      out_specs=pl.BlockSpec((1, BR, 128), lambda r, n, k: (n, r, 0)),
      out_shape=jax.ShapeDtypeStruct((nfb_out, N, 128), jnp.float32),
      compiler_params=pltpu.CompilerParams(
          dimension_semantics=("parallel", "parallel", "arbitrary")),
  )(seg, scl, b, w)


def _fin_body(seg_ref, scl_ref, b_ref, o_ref):
  sc = 1.0 / scl_ref[:, 0:1]
  t0 = seg_ref[0] * sc + b_ref[0]
  t1 = seg_ref[1] * sc + b_ref[1]
  ss = jnp.sum(t0 * t0 + t1 * t1, axis=1, keepdims=True)
  inv = 1.0 / jnp.maximum(jnp.sqrt(ss), 1e-12)
  o_ref[:, :128] = t0 * inv
  o_ref[:, 128:] = t1 * inv


def _fin(seg, scl, b):
  return pl.pallas_call(
      _fin_body,
      grid=(RB,),
      in_specs=[
          pl.BlockSpec((2, BR, 128), lambda r: (0, r, 0)),
          pl.BlockSpec((BR, 128), lambda r: (r, 0)),
          pl.BlockSpec((2, 128), lambda r: (0, 0)),
      ],
      out_specs=pl.BlockSpec((BR, 256), lambda r: (r, 0)),
      out_shape=jax.ShapeDtypeStruct((N, 256), jnp.float32),
      compiler_params=pltpu.CompilerParams(
          dimension_semantics=("parallel",)),
  )(seg, scl, b)


def kernel(x, rows, cols, vals, w0, b0, w1, b1, w2, b2, w3, b3, w4, b4):
  e = rows.shape[0]
  pad = EP - e
  cols_p = jnp.concatenate([cols.astype(jnp.int32),
                            jnp.zeros((pad,), jnp.int32)])
  rows_p = jnp.concatenate([rows.astype(jnp.int32),
                            jnp.full((pad,), N, jnp.int32)])
  idx = jnp.stack([rows_p.reshape(-1, W), cols_p.reshape(-1, W)], axis=1)
  zeros = jnp.zeros((ZROWS, 128), jnp.float32)
  ones = jnp.ones((W, 128), jnp.float32)

  hw = _mm0(x, w0)
  seg, scl = _spmm_first(hw, idx, zeros, ones)
  ws = [w1, w2, w3, w4]
  bs = [b0, b1, b2, b3]
  for i in range(4):
    nfb_out = 4 if i < 3 else 2
    hw = _mid(seg, scl, bs[i].reshape(4, 1, 128), ws[i], nfb_out)
    if i < 3:
      (seg,) = _spmm_mid(hw, idx, zeros)
    else:
      (seg,) = _spmm_last(hw, idx, zeros)
  return _fin(seg, scl, b4.reshape(2, 128))


# speedup vs baseline: 1.1544x; 1.1544x over previous
import functools

import jax
import jax.numpy as jnp
from jax import lax
from jax.experimental import pallas as pl
from jax.experimental.pallas import tpu as pltpu
from jax.experimental.pallas import tpu_sc as plsc

N = 10000
NPAD = 10240
W = 120
NSUB = 16
NCORE = 2
NWIN = 172
EDGES_PER_SUB = NWIN * W
EP = NSUB * EDGES_PER_SUB
RB = 10
BR = N // RB
ZROWS = 640
OROWS = 400

IB = 6
GB = 3
SB = 3
UNROLL = 6
PFD = 4
LEAD = 1
LAG = 2
CWIN = NWIN // 2


@functools.lru_cache(maxsize=None)
def _make_spmm(nfb, extract_scale):
  fpc = nfb // NCORE
  mesh = plsc.VectorSubcoreMesh(core_axis_name="c", subcore_axis_name="s",
                                num_cores=NCORE, num_subcores=NSUB)

  out_type = [jax.ShapeDtypeStruct((nfb, N, 128), jnp.float32)]
  if extract_scale:
    out_type.append(jax.ShapeDtypeStruct((2, N, 128), jnp.float32))

  scratch = (
      [pltpu.VMEM((2, W), jnp.int32) for _ in range(IB)] +
      [pltpu.VMEM((W, 128), jnp.float32) for _ in range(GB)] +
      [pltpu.VMEM_SHARED((NPAD, 128), jnp.float32)] +
      [pltpu.SemaphoreType.DMA for _ in range(IB + GB + SB)]
  )

  def body(hw, idxr, zerosr, *rest):
    if extract_scale:
      onesr, segr, cntr = rest[:3]
      rest = rest[3:]
    else:
      segr = rest[0]
      rest = rest[1:]
    idx_v = rest[:IB]
    g_v = rest[IB:IB + GB]
    acc_sh = rest[IB + GB]
    sem_i = rest[IB + GB + 1:IB + GB + 1 + IB]
    sem_g = rest[IB + GB + 1 + IB:IB + GB + 1 + IB + GB]
    sem_s = rest[IB + GB + 1 + IB + GB:]
    c = lax.axis_index("c")
    s = lax.axis_index("s")

    def idx_issue(w, m):
      pltpu.async_copy(idxr.at[s * NWIN + w], idx_v[m], sem_i[m])

    def idx_wait(w, m):
      pltpu.make_async_copy(idxr.at[s * NWIN + w], idx_v[m], sem_i[m]).wait()

    def writeout(dst):
      pltpu.sync_copy(acc_sh.at[pl.ds(s * OROWS, OROWS)],
                      dst.at[pl.ds(s * OROWS, OROWS)])

      @pl.when(s < 9)
      def _():
        pltpu.sync_copy(acc_sh.at[pl.ds((s + 16) * OROWS, OROWS)],
                        dst.at[pl.ds((s + 16) * OROWS, OROWS)])

    def run_pass(sc_issue, sc_wait, gather_issue, gather_wait, dst,
                 w0=0, nw=NWIN):
      pltpu.sync_copy(zerosr, acc_sh.at[pl.ds(s * ZROWS, ZROWS)])
      plsc.subcore_barrier()

      def bodyw(w, m, skip_scwait=False, do_idx=True, do_next=True):
        if not skip_scwait:
          sc_wait(w0 + w - LAG, (m - LAG) % IB, (m - LAG) % SB,
                  (m - LAG) % GB)
        if do_idx:
          idx_issue(w0 + w + PFD, (m + PFD) % IB)
        if do_next:
          idx_wait(w0 + w + LEAD, (m + LEAD) % IB)
          gather_issue(w0 + w + LEAD, (m + LEAD) % IB, (m + LEAD) % GB)
        gather_wait(w0 + w, m % IB, m % GB)
        sc_issue(w0 + w, m % IB, m % SB, m % GB)

      for w in range(PFD):
        idx_issue(w0 + w, w)
      for w in range(LEAD):
        idx_wait(w0 + w, w)
        gather_issue(w0 + w, w, w)
      for w in range(LAG):
        bodyw(w, w, skip_scwait=True)

      k_iters = (nw - LAG - PFD) // UNROLL
      tail_start = LAG + UNROLL * k_iters

      @pl.loop(LAG, tail_start, step=UNROLL)
      def _(t):
        for k in range(UNROLL):
          bodyw(t + k, (LAG + k) % UNROLL)

      for w in range(tail_start, nw):
        bodyw(w, w % UNROLL, do_idx=(w + PFD < nw),
              do_next=(w + LEAD < nw))
      for w in range(nw - LAG, nw):
        sc_wait(w0 + w, w % IB, w % SB, w % GB)

      plsc.subcore_barrier()
      writeout(dst)
      plsc.subcore_barrier()

    def mk_gather(fb):
      def gather_issue(w, m8, m4):
        pltpu.async_copy(hw.at[fb].at[idx_v[m8].at[1]], g_v[m4], sem_g[m4])

      def gather_wait(w, m8, m4):
        pltpu.make_async_copy(hw.at[fb].at[idx_v[m8].at[1]], g_v[m4],
                              sem_g[m4]).wait()

      def sc_issue(w, m8, msem, m4):
        pltpu.async_copy(g_v[m4], acc_sh.at[idx_v[m8].at[0]], sem_s[msem],
                         add=True)

      def sc_wait(w, m8, msem, m4):
        pltpu.make_async_copy(g_v[m4], acc_sh.at[idx_v[m8].at[0]],
                              sem_s[msem]).wait()

      return gather_issue, gather_wait, sc_issue, sc_wait

    if extract_scale:
      pltpu.sync_copy(onesr, g_v[0])

      def cnt_gather_issue(w, m8, m4):
        pass

      def cnt_gather_wait(w, m8, m4):
        pass

      def cnt_sc_issue(w, m8, msem, m4):
        pltpu.async_copy(g_v[0], acc_sh.at[idx_v[m8].at[0]], sem_s[msem],
                         add=True)

      def cnt_sc_wait(w, m8, msem, m4):
        pltpu.make_async_copy(g_v[0], acc_sh.at[idx_v[m8].at[0]],
                              sem_s[msem]).wait()

      run_pass(cnt_sc_issue, cnt_sc_wait, cnt_gather_issue, cnt_gather_wait,
               cntr.at[c], w0=c * CWIN, nw=CWIN)

    for j in range(fpc):
      fb = c * fpc + j
      gi, gw, si, sw = mk_gather(fb)
      run_pass(si, sw, gi, gw, segr.at[fb])

  return pl.kernel(body, out_type=tuple(out_type), mesh=mesh,
                   scratch_types=scratch)


def _spmm_first(*args):
  return _make_spmm(4, True)(*args)


def _spmm_mid(*args):
  return _make_spmm(4, False)(*args)


def _spmm_last(*args):
  return _make_spmm(2, False)(*args)


def _mm0_body(x_ref, w_ref, o_ref):
  o_ref[0] = jnp.dot(x_ref[...], w_ref[...],
                     preferred_element_type=jnp.float32)


def _mm0(x, w):
  return pl.pallas_call(
      _mm0_body,
      grid=(RB, 4),
      in_specs=[
          pl.BlockSpec((BR, 256), lambda r, n: (r, 0)),
          pl.BlockSpec((256, 128), lambda r, n: (0, n)),
      ],
      out_specs=pl.BlockSpec((1, BR, 128), lambda r, n: (n, r, 0)),
      out_shape=jax.ShapeDtypeStruct((4, N, 128), jnp.float32),
      compiler_params=pltpu.CompilerParams(
          dimension_semantics=("parallel", "parallel")),
  )(x, w)


def _mid_body(seg_ref, scl_ref, b_ref, w_ref, o_ref):
  k = pl.program_id(2)
  t = seg_ref[0] * (1.0 / (scl_ref[0, :, 0:1] + scl_ref[1, :, 0:1])) + b_ref[0, 0]
  t = jnp.where(t >= 0, t, 0.2 * t)
  p = jnp.dot(t, w_ref[...], preferred_element_type=jnp.float32)

  @pl.when(k == 0)
  def _():
    o_ref[0] = p

  @pl.when(k > 0)
  def _():
    o_ref[0] += p


def _mid(seg, scl, b, w, nfb_out):
  nfb_in = seg.shape[0]
  return pl.pallas_call(
      _mid_body,
      grid=(RB, nfb_out, nfb_in),
      in_specs=[
          pl.BlockSpec((1, BR, 128), lambda r, n, k: (k, r, 0)),
          pl.BlockSpec((2, BR, 128), lambda r, n, k: (0, r, 0)),
          pl.BlockSpec((1, 1, 128), lambda r, n, k: (k, 0, 0)),
          pl.BlockSpec((128, 128), lambda r, n, k: (k, n)),
      ],
      out_specs=pl.BlockSpec((1, BR, 128), lambda r, n, k: (n, r, 0)),
      out_shape=jax.ShapeDtypeStruct((nfb_out, N, 128), jnp.float32),
      compiler_params=pltpu.CompilerParams(
          dimension_semantics=("parallel", "parallel", "arbitrary")),
  )(seg, scl, b, w)


def _fin_body(seg_ref, scl_ref, b_ref, o_ref):
  sc = 1.0 / (scl_ref[0, :, 0:1] + scl_ref[1, :, 0:1])
  t0 = seg_ref[0] * sc + b_ref[0]
  t1 = seg_ref[1] * sc + b_ref[1]
  ss = jnp.sum(t0 * t0 + t1 * t1, axis=1, keepdims=True)
  inv = 1.0 / jnp.maximum(jnp.sqrt(ss), 1e-12)
  o_ref[:, :128] = t0 * inv
  o_ref[:, 128:] = t1 * inv


def _fin(seg, scl, b):
  return pl.pallas_call(
      _fin_body,
      grid=(RB,),
      in_specs=[
          pl.BlockSpec((2, BR, 128), lambda r: (0, r, 0)),
          pl.BlockSpec((2, BR, 128), lambda r: (0, r, 0)),
          pl.BlockSpec((2, 128), lambda r: (0, 0)),
      ],
      out_specs=pl.BlockSpec((BR, 256), lambda r: (r, 0)),
      out_shape=jax.ShapeDtypeStruct((N, 256), jnp.float32),
      compiler_params=pltpu.CompilerParams(
          dimension_semantics=("parallel",)),
  )(seg, scl, b)


def kernel(x, rows, cols, vals, w0, b0, w1, b1, w2, b2, w3, b3, w4, b4):
  e = rows.shape[0]
  pad = EP - e
  cols_p = jnp.concatenate([cols.astype(jnp.int32),
                            jnp.zeros((pad,), jnp.int32)])
  rows_p = jnp.concatenate([rows.astype(jnp.int32),
                            jnp.full((pad,), N, jnp.int32)])
  idx = jnp.stack([rows_p.reshape(-1, W), cols_p.reshape(-1, W)], axis=1)
  zeros = jnp.zeros((ZROWS, 128), jnp.float32)
  ones = jnp.ones((W, 128), jnp.float32)

  hw = _mm0(x, w0)
  seg, scl = _spmm_first(hw, idx, zeros, ones)
  ws = [w1, w2, w3, w4]
  bs = [b0, b1, b2, b3]
  for i in range(4):
    nfb_out = 4 if i < 3 else 2
    hw = _mid(seg, scl, bs[i].reshape(4, 1, 128), ws[i], nfb_out)
    if i < 3:
      (seg,) = _spmm_mid(hw, idx, zeros)
    else:
      (seg,) = _spmm_last(hw, idx, zeros)
  return _fin(seg, scl, b4.reshape(2, 128))


# speedup vs baseline: 1.2954x; 1.1221x over previous
import functools

import jax
import jax.numpy as jnp
from jax import lax
from jax.experimental import pallas as pl
from jax.experimental.pallas import tpu as pltpu
from jax.experimental.pallas import tpu_sc as plsc

N = 10000
NPAD = 10240
W = 120
NSUB = 16
NCORE = 2
NWIN = 172
EDGES_PER_SUB = NWIN * W
EP = NSUB * EDGES_PER_SUB
RB = 10
BR = N // RB
ZROWS = 640
OROWS = 400

IB = 6
GB = 3
SB = 3
UNROLL = 6
PFD = 4
LEAD = 1
LAG = 2
CWIN = NWIN // 2


@functools.lru_cache(maxsize=None)
def _make_spmm(nfb, extract_scale):
  fpc = nfb // NCORE
  mesh = plsc.VectorSubcoreMesh(core_axis_name="c", subcore_axis_name="s",
                                num_cores=NCORE, num_subcores=NSUB)

  out_type = [jax.ShapeDtypeStruct((nfb, N, 128), jnp.float32)]
  if extract_scale:
    out_type.append(jax.ShapeDtypeStruct((2, N, 128), jnp.float32))

  scratch = (
      [pltpu.VMEM((2, W), jnp.int32) for _ in range(IB)] +
      [pltpu.VMEM((W, 128), jnp.float32) for _ in range(GB)] +
      [pltpu.VMEM_SHARED((NPAD, 128), jnp.float32)] +
      [pltpu.SemaphoreType.DMA for _ in range(IB + GB + SB)]
  )

  def body(hw, idxr, zerosr, *rest):
    if extract_scale:
      onesr, segr, cntr = rest[:3]
      rest = rest[3:]
    else:
      segr = rest[0]
      rest = rest[1:]
    idx_v = rest[:IB]
    g_v = rest[IB:IB + GB]
    acc_sh = rest[IB + GB]
    sem_i = rest[IB + GB + 1:IB + GB + 1 + IB]
    sem_g = rest[IB + GB + 1 + IB:IB + GB + 1 + IB + GB]
    sem_s = rest[IB + GB + 1 + IB + GB:]
    c = lax.axis_index("c")
    s = lax.axis_index("s")

    def idx_issue(w, m):
      pltpu.async_copy(idxr.at[s * NWIN + w], idx_v[m], sem_i[m])

    def idx_wait(w, m):
      pltpu.make_async_copy(idxr.at[s * NWIN + w], idx_v[m], sem_i[m]).wait()

    def writeout(dst):
      pltpu.sync_copy(acc_sh.at[pl.ds(s * OROWS, OROWS)],
                      dst.at[pl.ds(s * OROWS, OROWS)])

      @pl.when(s < 9)
      def _():
        pltpu.sync_copy(acc_sh.at[pl.ds((s + 16) * OROWS, OROWS)],
                        dst.at[pl.ds((s + 16) * OROWS, OROWS)])

    def run_pass(sc_issue, sc_wait, gather_issue, gather_wait, dst,
                 w0=0, nw=NWIN):
      pltpu.sync_copy(zerosr, acc_sh.at[pl.ds(s * ZROWS, ZROWS)])
      plsc.subcore_barrier()

      def bodyw(w, m, skip_scwait=False, do_idx=True, do_next=True):
        if not skip_scwait:
          sc_wait(w0 + w - LAG, (m - LAG) % IB, (m - LAG) % SB,
                  (m - LAG) % GB)
        if do_idx:
          idx_issue(w0 + w + PFD, (m + PFD) % IB)
        if do_next:
          idx_wait(w0 + w + LEAD, (m + LEAD) % IB)
          gather_issue(w0 + w + LEAD, (m + LEAD) % IB, (m + LEAD) % GB)
        gather_wait(w0 + w, m % IB, m % GB)
        sc_issue(w0 + w, m % IB, m % SB, m % GB)

      for w in range(PFD):
        idx_issue(w0 + w, w)
      for w in range(LEAD):
        idx_wait(w0 + w, w)
        gather_issue(w0 + w, w, w)
      for w in range(LAG):
        bodyw(w, w, skip_scwait=True)

      k_iters = (nw - LAG - PFD) // UNROLL
      tail_start = LAG + UNROLL * k_iters

      @pl.loop(LAG, tail_start, step=UNROLL)
      def _(t):
        for k in range(UNROLL):
          bodyw(t + k, (LAG + k) % UNROLL)

      for w in range(tail_start, nw):
        bodyw(w, w % UNROLL, do_idx=(w + PFD < nw),
              do_next=(w + LEAD < nw))
      for w in range(nw - LAG, nw):
        sc_wait(w0 + w, w % IB, w % SB, w % GB)

      plsc.subcore_barrier()
      writeout(dst)
      plsc.subcore_barrier()

    def mk_gather(fb):
      def gather_issue(w, m8, m4):
        pltpu.async_copy(hw.at[fb].at[idx_v[m8].at[1]], g_v[m4], sem_g[m4])

      def gather_wait(w, m8, m4):
        pltpu.make_async_copy(hw.at[fb].at[idx_v[m8].at[1]], g_v[m4],
                              sem_g[m4]).wait()

      def sc_issue(w, m8, msem, m4):
        pltpu.async_copy(g_v[m4], acc_sh.at[idx_v[m8].at[0]], sem_s[msem],
                         add=True)

      def sc_wait(w, m8, msem, m4):
        pltpu.make_async_copy(g_v[m4], acc_sh.at[idx_v[m8].at[0]],
                              sem_s[msem]).wait()

      return gather_issue, gather_wait, sc_issue, sc_wait

    if extract_scale:
      pltpu.sync_copy(onesr, g_v[0])

      def cnt_gather_issue(w, m8, m4):
        pass

      def cnt_gather_wait(w, m8, m4):
        pass

      def cnt_sc_issue(w, m8, msem, m4):
        pltpu.async_copy(g_v[0], acc_sh.at[idx_v[m8].at[0]], sem_s[msem],
                         add=True)

      def cnt_sc_wait(w, m8, msem, m4):
        pltpu.make_async_copy(g_v[0], acc_sh.at[idx_v[m8].at[0]],
                              sem_s[msem]).wait()

      run_pass(cnt_sc_issue, cnt_sc_wait, cnt_gather_issue, cnt_gather_wait,
               cntr.at[c], w0=c * CWIN, nw=CWIN)

    for j in range(fpc):
      fb = c * fpc + j
      gi, gw, si, sw = mk_gather(fb)
      run_pass(si, sw, gi, gw, segr.at[fb])

  return pl.kernel(body, out_type=tuple(out_type), mesh=mesh,
                   scratch_types=scratch)


def _spmm_first(*args):
  return _make_spmm(2, True)(*args)


def _spmm_mid(*args):
  return _make_spmm(2, False)(*args)


def _spmm_last(*args):
  return _make_spmm(2, False)(*args)


def _mm0_body(x_ref, w_ref, o_ref):
  o_ref[0] = jnp.dot(x_ref[...], w_ref[...],
                     preferred_element_type=jnp.float32)


def _mm0h(x, wh):
  return pl.pallas_call(
      _mm0_body,
      grid=(RB, 2),
      in_specs=[
          pl.BlockSpec((BR, 256), lambda r, n: (r, 0)),
          pl.BlockSpec((256, 128), lambda r, n: (0, n)),
      ],
      out_specs=pl.BlockSpec((1, BR, 128), lambda r, n: (n, r, 0)),
      out_shape=jax.ShapeDtypeStruct((2, N, 128), jnp.float32),
      compiler_params=pltpu.CompilerParams(
          dimension_semantics=("parallel", "parallel")),
  )(x, wh)


def _act(seg_ref, scl_ref, b_ref):
  t = (seg_ref[0] * (1.0 / (scl_ref[0, :, 0:1] + scl_ref[1, :, 0:1]))
       + b_ref[0, 0])
  return jnp.where(t >= 0, t, 0.2 * t)


def _midp_body(seg_ref, scl_ref, b_ref, w_ref, o_ref):
  k = pl.program_id(2)
  p = jnp.dot(_act(seg_ref, scl_ref, b_ref), w_ref[...],
              preferred_element_type=jnp.float32)

  @pl.when(k == 0)
  def _():
    o_ref[0] = p

  @pl.when(k > 0)
  def _():
    o_ref[0] += p


def _midp_acc_body(seg_ref, scl_ref, b_ref, w_ref, a_ref, o_ref):
  k = pl.program_id(2)
  p = jnp.dot(_act(seg_ref, scl_ref, b_ref), w_ref[...],
              preferred_element_type=jnp.float32)

  @pl.when(k == 0)
  def _():
    o_ref[0] = a_ref[0] + p

  @pl.when(k > 0)
  def _():
    o_ref[0] += p


def _midp(seg, scl, b2, wq, acc, n_out):
  in_specs = [
      pl.BlockSpec((1, BR, 128), lambda r, n, k: (k, r, 0)),
      pl.BlockSpec((2, BR, 128), lambda r, n, k: (0, r, 0)),
      pl.BlockSpec((1, 1, 128), lambda r, n, k: (k, 0, 0)),
      pl.BlockSpec((128, 128), lambda r, n, k: (k, n)),
  ]
  args = [seg, scl, b2, wq]
  body = _midp_body
  if acc is not None:
    in_specs.append(pl.BlockSpec((1, BR, 128), lambda r, n, k: (n, r, 0)))
    args.append(acc)
    body = _midp_acc_body
  return pl.pallas_call(
      body,
      grid=(RB, n_out, 2),
      in_specs=in_specs,
      out_specs=pl.BlockSpec((1, BR, 128), lambda r, n, k: (n, r, 0)),
      out_shape=jax.ShapeDtypeStruct((n_out, N, 128), jnp.float32),
      compiler_params=pltpu.CompilerParams(
          dimension_semantics=("parallel", "parallel", "arbitrary")),
  )(*args)


def _fin_body(seg_ref, scl_ref, b_ref, o_ref):
  sc = 1.0 / (scl_ref[0, :, 0:1] + scl_ref[1, :, 0:1])
  t0 = seg_ref[0] * sc + b_ref[0]
  t1 = seg_ref[1] * sc + b_ref[1]
  ss = jnp.sum(t0 * t0 + t1 * t1, axis=1, keepdims=True)
  inv = 1.0 / jnp.maximum(jnp.sqrt(ss), 1e-12)
  o_ref[:, :128] = t0 * inv
  o_ref[:, 128:] = t1 * inv


def _fin(seg, scl, b):
  return pl.pallas_call(
      _fin_body,
      grid=(RB,),
      in_specs=[
          pl.BlockSpec((2, BR, 128), lambda r: (0, r, 0)),
          pl.BlockSpec((2, BR, 128), lambda r: (0, r, 0)),
          pl.BlockSpec((2, 128), lambda r: (0, 0)),
      ],
      out_specs=pl.BlockSpec((BR, 256), lambda r: (r, 0)),
      out_shape=jax.ShapeDtypeStruct((N, 256), jnp.float32),
      compiler_params=pltpu.CompilerParams(
          dimension_semantics=("parallel",)),
  )(seg, scl, b)


def kernel(x, rows, cols, vals, w0, b0, w1, b1, w2, b2, w3, b3, w4, b4):
  e = rows.shape[0]
  pad = EP - e
  cols_p = jnp.concatenate([cols.astype(jnp.int32),
                            jnp.zeros((pad,), jnp.int32)])
  rows_p = jnp.concatenate([rows.astype(jnp.int32),
                            jnp.full((pad,), N, jnp.int32)])
  idx = jnp.stack([rows_p.reshape(-1, W), cols_p.reshape(-1, W)], axis=1)
  zeros = jnp.zeros((ZROWS, 128), jnp.float32)
  ones = jnp.ones((W, 128), jnp.float32)

  hw_a = _mm0h(x, w0[:, :256])
  hw_b = _mm0h(x, w0[:, 256:])
  seg_a, scl = _spmm_first(hw_a, idx, zeros, ones)
  (seg_b,) = _spmm_mid(hw_b, idx, zeros)
  ws = [w1, w2, w3]
  bs = [b0, b1, b2, b3]
  for i in range(3):
    b2a = bs[i].reshape(4, 1, 128)[0:2]
    b2b = bs[i].reshape(4, 1, 128)[2:4]
    wi = ws[i]
    p_a = _midp(seg_a, scl, b2a, wi[:256, :256], None, 2)
    p_b = _midp(seg_a, scl, b2a, wi[:256, 256:], None, 2)
    hw_a = _midp(seg_b, scl, b2b, wi[256:, :256], p_a, 2)
    hw_b = _midp(seg_b, scl, b2b, wi[256:, 256:], p_b, 2)
    (seg_a,) = _spmm_mid(hw_a, idx, zeros)
    (seg_b,) = _spmm_mid(hw_b, idx, zeros)
  b2a = b3.reshape(4, 1, 128)[0:2]
  b2b = b3.reshape(4, 1, 128)[2:4]
  p_a = _midp(seg_a, scl, b2a, w4[:256, :128], None, 1)
  p_b = _midp(seg_a, scl, b2a, w4[:256, 128:], None, 1)
  hw4_a = _midp(seg_b, scl, b2b, w4[256:, :128], p_a, 1)
  hw4_b = _midp(seg_b, scl, b2b, w4[256:, 128:], p_b, 1)
  hw4 = jnp.concatenate([hw4_a, hw4_b], axis=0)
  (seg,) = _spmm_mid(hw4, idx, zeros)
  return _fin(seg, scl, b4.reshape(2, 128))


# speedup vs baseline: 1.2997x; 1.0034x over previous
import functools

import jax
import jax.numpy as jnp
from jax import lax
from jax.experimental import pallas as pl
from jax.experimental.pallas import tpu as pltpu
from jax.experimental.pallas import tpu_sc as plsc

N = 10000
NPAD = 10240
W = 120
NSUB = 16
NCORE = 2
NWIN = 172
EDGES_PER_SUB = NWIN * W
EP = NSUB * EDGES_PER_SUB
RB = 10
BR = N // RB
ZROWS = 640
OROWS = 400

IB = 6
GB = 3
SB = 3
UNROLL = 6
PFD = 4
LEAD = 1
LAG = 2
CWIN = NWIN // 2


@functools.lru_cache(maxsize=None)
def _make_spmm(nfb, extract_scale):
  fpc = nfb // NCORE
  mesh = plsc.VectorSubcoreMesh(core_axis_name="c", subcore_axis_name="s",
                                num_cores=NCORE, num_subcores=NSUB)

  out_type = [jax.ShapeDtypeStruct((nfb, N, 128), jnp.float32)]
  if extract_scale:
    out_type.append(jax.ShapeDtypeStruct((2, N, 128), jnp.float32))

  scratch = (
      [pltpu.VMEM((2, W), jnp.int32) for _ in range(IB)] +
      [pltpu.VMEM((W, 128), jnp.float32) for _ in range(GB)] +
      [pltpu.VMEM_SHARED((NPAD, 128), jnp.float32)] +
      [pltpu.SemaphoreType.DMA for _ in range(IB + GB + SB)]
  )

  def body(hw, idxr, zerosr, *rest):
    if extract_scale:
      onesr, segr, cntr = rest[:3]
      rest = rest[3:]
    else:
      segr = rest[0]
      rest = rest[1:]
    idx_v = rest[:IB]
    g_v = rest[IB:IB + GB]
    acc_sh = rest[IB + GB]
    sem_i = rest[IB + GB + 1:IB + GB + 1 + IB]
    sem_g = rest[IB + GB + 1 + IB:IB + GB + 1 + IB + GB]
    sem_s = rest[IB + GB + 1 + IB + GB:]
    c = lax.axis_index("c")
    s = lax.axis_index("s")

    def idx_issue(w, m):
      pltpu.async_copy(idxr.at[s * NWIN + w], idx_v[m], sem_i[m])

    def idx_wait(w, m):
      pltpu.make_async_copy(idxr.at[s * NWIN + w], idx_v[m], sem_i[m]).wait()

    def writeout(dst):
      pltpu.sync_copy(acc_sh.at[pl.ds(s * OROWS, OROWS)],
                      dst.at[pl.ds(s * OROWS, OROWS)])

      @pl.when(s < 9)
      def _():
        pltpu.sync_copy(acc_sh.at[pl.ds((s + 16) * OROWS, OROWS)],
                        dst.at[pl.ds((s + 16) * OROWS, OROWS)])

    def run_pass(sc_issue, sc_wait, gather_issue, gather_wait, dst,
                 w0=0, nw=NWIN):
      pltpu.sync_copy(zerosr, acc_sh.at[pl.ds(s * ZROWS, ZROWS)])
      plsc.subcore_barrier()

      def bodyw(w, m, skip_scwait=False, do_idx=True, do_next=True):
        if not skip_scwait:
          sc_wait(w0 + w - LAG, (m - LAG) % IB, (m - LAG) % SB,
                  (m - LAG) % GB)
        if do_idx:
          idx_issue(w0 + w + PFD, (m + PFD) % IB)
        if do_next:
          idx_wait(w0 + w + LEAD, (m + LEAD) % IB)
          gather_issue(w0 + w + LEAD, (m + LEAD) % IB, (m + LEAD) % GB)
        gather_wait(w0 + w, m % IB, m % GB)
        sc_issue(w0 + w, m % IB, m % SB, m % GB)

      for w in range(PFD):
        idx_issue(w0 + w, w)
      for w in range(LEAD):
        idx_wait(w0 + w, w)
        gather_issue(w0 + w, w, w)
      for w in range(LAG):
        bodyw(w, w, skip_scwait=True)

      k_iters = (nw - LAG - PFD) // UNROLL
      tail_start = LAG + UNROLL * k_iters

      @pl.loop(LAG, tail_start, step=UNROLL)
      def _(t):
        for k in range(UNROLL):
          bodyw(t + k, (LAG + k) % UNROLL)

      for w in range(tail_start, nw):
        bodyw(w, w % UNROLL, do_idx=(w + PFD < nw),
              do_next=(w + LEAD < nw))
      for w in range(nw - LAG, nw):
        sc_wait(w0 + w, w % IB, w % SB, w % GB)

      plsc.subcore_barrier()
      writeout(dst)
      plsc.subcore_barrier()

    def mk_gather(fb):
      def gather_issue(w, m8, m4):
        pltpu.async_copy(hw.at[fb].at[idx_v[m8].at[1]], g_v[m4], sem_g[m4])

      def gather_wait(w, m8, m4):
        pltpu.make_async_copy(hw.at[fb].at[idx_v[m8].at[1]], g_v[m4],
                              sem_g[m4]).wait()

      def sc_issue(w, m8, msem, m4):
        pltpu.async_copy(g_v[m4], acc_sh.at[idx_v[m8].at[0]], sem_s[msem],
                         add=True)

      def sc_wait(w, m8, msem, m4):
        pltpu.make_async_copy(g_v[m4], acc_sh.at[idx_v[m8].at[0]],
                              sem_s[msem]).wait()

      return gather_issue, gather_wait, sc_issue, sc_wait

    if extract_scale:
      pltpu.sync_copy(onesr, g_v[0])

      def cnt_gather_issue(w, m8, m4):
        pass

      def cnt_gather_wait(w, m8, m4):
        pass

      def cnt_sc_issue(w, m8, msem, m4):
        pltpu.async_copy(g_v[0], acc_sh.at[idx_v[m8].at[0]], sem_s[msem],
                         add=True)

      def cnt_sc_wait(w, m8, msem, m4):
        pltpu.make_async_copy(g_v[0], acc_sh.at[idx_v[m8].at[0]],
                              sem_s[msem]).wait()

      run_pass(cnt_sc_issue, cnt_sc_wait, cnt_gather_issue, cnt_gather_wait,
               cntr.at[c], w0=c * CWIN, nw=CWIN)

    for j in range(fpc):
      fb = c * fpc + j
      gi, gw, si, sw = mk_gather(fb)
      run_pass(si, sw, gi, gw, segr.at[fb])

  return pl.kernel(body, out_type=tuple(out_type), mesh=mesh,
                   scratch_types=scratch)


def _spmm_first(*args):
  return _make_spmm(2, True)(*args)


def _spmm_mid(*args):
  return _make_spmm(2, False)(*args)


def _spmm_last(*args):
  return _make_spmm(2, False)(*args)


def _mm0_body(x_ref, w_ref, o_ref):
  o_ref[0] = jnp.dot(x_ref[...], w_ref[...],
                     preferred_element_type=jnp.float32)


def _mm0h(x, wh):
  return pl.pallas_call(
      _mm0_body,
      grid=(RB, 2),
      in_specs=[
          pl.BlockSpec((BR, 256), lambda r, n: (r, 0)),
          pl.BlockSpec((256, 128), lambda r, n: (0, n)),
      ],
      out_specs=pl.BlockSpec((1, BR, 128), lambda r, n: (n, r, 0)),
      out_shape=jax.ShapeDtypeStruct((2, N, 128), jnp.float32),
      compiler_params=pltpu.CompilerParams(
          dimension_semantics=("parallel", "parallel")),
  )(x, wh)


def _act(seg_ref, scl_ref, b_ref):
  t = (seg_ref[0] * (1.0 / (scl_ref[0, :, 0:1] + scl_ref[1, :, 0:1]))
       + b_ref[0, 0])
  return jnp.where(t >= 0, t, 0.2 * t)


def _midp_body(seg_ref, scl_ref, b_ref, w_ref, o_ref):
  k = pl.program_id(2)
  p = jnp.dot(_act(seg_ref, scl_ref, b_ref), w_ref[...],
              preferred_element_type=jnp.float32)

  @pl.when(k == 0)
  def _():
    o_ref[0] = p

  @pl.when(k > 0)
  def _():
    o_ref[0] += p


def _midp_acc_body(seg_ref, scl_ref, b_ref, w_ref, a_ref, o_ref):
  k = pl.program_id(2)
  p = jnp.dot(_act(seg_ref, scl_ref, b_ref), w_ref[...],
              preferred_element_type=jnp.float32)

  @pl.when(k == 0)
  def _():
    o_ref[0] = a_ref[0] + p

  @pl.when(k > 0)
  def _():
    o_ref[0] += p


def _midp(seg, scl, b2, wq, acc, n_out, noff=0):
  in_specs = [
      pl.BlockSpec((1, BR, 128), lambda r, n, k: (k, r, 0)),
      pl.BlockSpec((2, BR, 128), lambda r, n, k: (0, r, 0)),
      pl.BlockSpec((1, 1, 128), lambda r, n, k: (k, 0, 0)),
      pl.BlockSpec((128, 128), lambda r, n, k: (k, n)),
  ]
  args = [seg, scl, b2, wq]
  body = _midp_body
  if acc is not None:
    in_specs.append(
        pl.BlockSpec((1, BR, 128), lambda r, n, k, o=noff: (n + o, r, 0)))
    args.append(acc)
    body = _midp_acc_body
  return pl.pallas_call(
      body,
      grid=(RB, n_out, 2),
      in_specs=in_specs,
      out_specs=pl.BlockSpec((1, BR, 128), lambda r, n, k: (n, r, 0)),
      out_shape=jax.ShapeDtypeStruct((n_out, N, 128), jnp.float32),
      compiler_params=pltpu.CompilerParams(
          dimension_semantics=("parallel", "parallel", "arbitrary")),
  )(*args)


def _fin_body(seg_ref, scl_ref, b_ref, o_ref):
  sc = 1.0 / (scl_ref[0, :, 0:1] + scl_ref[1, :, 0:1])
  t0 = seg_ref[0] * sc + b_ref[0]
  t1 = seg_ref[1] * sc + b_ref[1]
  ss = jnp.sum(t0 * t0 + t1 * t1, axis=1, keepdims=True)
  inv = 1.0 / jnp.maximum(jnp.sqrt(ss), 1e-12)
  o_ref[:, :128] = t0 * inv
  o_ref[:, 128:] = t1 * inv


def _fin(seg, scl, b):
  return pl.pallas_call(
      _fin_body,
      grid=(RB,),
      in_specs=[
          pl.BlockSpec((2, BR, 128), lambda r: (0, r, 0)),
          pl.BlockSpec((2, BR, 128), lambda r: (0, r, 0)),
          pl.BlockSpec((2, 128), lambda r: (0, 0)),
      ],
      out_specs=pl.BlockSpec((BR, 256), lambda r: (r, 0)),
      out_shape=jax.ShapeDtypeStruct((N, 256), jnp.float32),
      compiler_params=pltpu.CompilerParams(
          dimension_semantics=("parallel",)),
  )(seg, scl, b)


def kernel(x, rows, cols, vals, w0, b0, w1, b1, w2, b2, w3, b3, w4, b4):
  e = rows.shape[0]
  pad = EP - e
  cols_p = jnp.concatenate([cols.astype(jnp.int32),
                            jnp.zeros((pad,), jnp.int32)])
  rows_p = jnp.concatenate([rows.astype(jnp.int32),
                            jnp.full((pad,), N, jnp.int32)])
  idx = jnp.stack([rows_p.reshape(-1, W), cols_p.reshape(-1, W)], axis=1)
  zeros = jnp.zeros((ZROWS, 128), jnp.float32)
  ones = jnp.ones((W, 128), jnp.float32)

  hw_a = _mm0h(x, w0[:, :256])
  hw_b = _mm0h(x, w0[:, 256:])
  seg_a, scl = _spmm_first(hw_a, idx, zeros, ones)
  (seg_b,) = _spmm_mid(hw_b, idx, zeros)
  ws = [w1, w2, w3]
  bs = [b0, b1, b2, b3]
  for i in range(3):
    b2a = bs[i].reshape(4, 1, 128)[0:2]
    b2b = bs[i].reshape(4, 1, 128)[2:4]
    wi = ws[i]
    p = _midp(seg_a, scl, b2a, wi[:256, :], None, 4)
    hw_a = _midp(seg_b, scl, b2b, wi[256:, :256], p, 2, noff=0)
    hw_b = _midp(seg_b, scl, b2b, wi[256:, 256:], p, 2, noff=2)
    (seg_a,) = _spmm_mid(hw_a, idx, zeros)
    (seg_b,) = _spmm_mid(hw_b, idx, zeros)
  b2a = b3.reshape(4, 1, 128)[0:2]
  b2b = b3.reshape(4, 1, 128)[2:4]
  p = _midp(seg_a, scl, b2a, w4[:256, :], None, 2)
  hw4_a = _midp(seg_b, scl, b2b, w4[256:, :128], p, 1, noff=0)
  hw4_b = _midp(seg_b, scl, b2b, w4[256:, 128:], p, 1, noff=1)
  hw4 = jnp.concatenate([hw4_a, hw4_b], axis=0)
  (seg,) = _spmm_mid(hw4, idx, zeros)
  return _fin(seg, scl, b4.reshape(2, 128))


# speedup vs baseline: 1.3069x; 1.0055x over previous
import functools

import jax
import jax.numpy as jnp
from jax import lax
from jax.experimental import pallas as pl
from jax.experimental.pallas import tpu as pltpu
from jax.experimental.pallas import tpu_sc as plsc

N = 10000
NPAD = 10240
W = 120
NSUB = 16
NCORE = 2
NWIN = 172
EDGES_PER_SUB = NWIN * W
EP = NSUB * EDGES_PER_SUB
RB = 10
BR = N // RB
ZROWS = 640
OROWS = 400

IB = 6
GB = 3
SB = 3
UNROLL = 6
PFD = 4
LEAD = 1
LAG = 2
CWIN = NWIN // 2


@functools.lru_cache(maxsize=None)
def _make_spmm(nfb, extract_scale):
  fpc = nfb // NCORE
  mesh = plsc.VectorSubcoreMesh(core_axis_name="c", subcore_axis_name="s",
                                num_cores=NCORE, num_subcores=NSUB)

  out_type = [jax.ShapeDtypeStruct((nfb, N, 128), jnp.float32)]
  if extract_scale:
    out_type.append(jax.ShapeDtypeStruct((2, N, 128), jnp.float32))

  scratch = (
      [pltpu.VMEM((2, W), jnp.int32) for _ in range(IB)] +
      [pltpu.VMEM((W, 128), jnp.float32) for _ in range(GB)] +
      [pltpu.VMEM_SHARED((NPAD, 128), jnp.float32)] +
      [pltpu.SemaphoreType.DMA for _ in range(IB + GB + SB)]
  )

  def body(hw, idxr, zerosr, *rest):
    if extract_scale:
      onesr, segr, cntr = rest[:3]
      rest = rest[3:]
    else:
      segr = rest[0]
      rest = rest[1:]
    idx_v = rest[:IB]
    g_v = rest[IB:IB + GB]
    acc_sh = rest[IB + GB]
    sem_i = rest[IB + GB + 1:IB + GB + 1 + IB]
    sem_g = rest[IB + GB + 1 + IB:IB + GB + 1 + IB + GB]
    sem_s = rest[IB + GB + 1 + IB + GB:]
    c = lax.axis_index("c")
    s = lax.axis_index("s")

    def idx_issue(w, m):
      pltpu.async_copy(idxr.at[s * NWIN + w], idx_v[m], sem_i[m])

    def idx_wait(w, m):
      pltpu.make_async_copy(idxr.at[s * NWIN + w], idx_v[m], sem_i[m]).wait()

    def writeout(dst):
      pltpu.sync_copy(acc_sh.at[pl.ds(s * OROWS, OROWS)],
                      dst.at[pl.ds(s * OROWS, OROWS)])

      @pl.when(s < 9)
      def _():
        pltpu.sync_copy(acc_sh.at[pl.ds((s + 16) * OROWS, OROWS)],
                        dst.at[pl.ds((s + 16) * OROWS, OROWS)])

    def run_pass(sc_issue, sc_wait, gather_issue, gather_wait, dst,
                 w0=0, nw=NWIN):
      pltpu.sync_copy(zerosr, acc_sh.at[pl.ds(s * ZROWS, ZROWS)])
      plsc.subcore_barrier()

      def bodyw(w, m, skip_scwait=False, do_idx=True, do_next=True):
        if not skip_scwait:
          sc_wait(w0 + w - LAG, (m - LAG) % IB, (m - LAG) % SB,
                  (m - LAG) % GB)
        if do_idx:
          idx_issue(w0 + w + PFD, (m + PFD) % IB)
        if do_next:
          idx_wait(w0 + w + LEAD, (m + LEAD) % IB)
          gather_issue(w0 + w + LEAD, (m + LEAD) % IB, (m + LEAD) % GB)
        gather_wait(w0 + w, m % IB, m % GB)
        sc_issue(w0 + w, m % IB, m % SB, m % GB)

      for w in range(PFD):
        idx_issue(w0 + w, w)
      for w in range(LEAD):
        idx_wait(w0 + w, w)
        gather_issue(w0 + w, w, w)
      for w in range(LAG):
        bodyw(w, w, skip_scwait=True)

      k_iters = (nw - LAG - PFD) // UNROLL
      tail_start = LAG + UNROLL * k_iters

      @pl.loop(LAG, tail_start, step=UNROLL)
      def _(t):
        for k in range(UNROLL):
          bodyw(t + k, (LAG + k) % UNROLL)

      for w in range(tail_start, nw):
        bodyw(w, w % UNROLL, do_idx=(w + PFD < nw),
              do_next=(w + LEAD < nw))
      for w in range(nw - LAG, nw):
        sc_wait(w0 + w, w % IB, w % SB, w % GB)

      plsc.subcore_barrier()
      writeout(dst)
      plsc.subcore_barrier()

    def mk_gather(fb):
      def gather_issue(w, m8, m4):
        pltpu.async_copy(hw.at[fb].at[idx_v[m8].at[1]], g_v[m4], sem_g[m4])

      def gather_wait(w, m8, m4):
        pltpu.make_async_copy(hw.at[fb].at[idx_v[m8].at[1]], g_v[m4],
                              sem_g[m4]).wait()

      def sc_issue(w, m8, msem, m4):
        pltpu.async_copy(g_v[m4], acc_sh.at[idx_v[m8].at[0]], sem_s[msem],
                         add=True)

      def sc_wait(w, m8, msem, m4):
        pltpu.make_async_copy(g_v[m4], acc_sh.at[idx_v[m8].at[0]],
                              sem_s[msem]).wait()

      return gather_issue, gather_wait, sc_issue, sc_wait

    if extract_scale:
      pltpu.sync_copy(onesr, g_v[0])

      def cnt_gather_issue(w, m8, m4):
        pass

      def cnt_gather_wait(w, m8, m4):
        pass

      def cnt_sc_issue(w, m8, msem, m4):
        pltpu.async_copy(g_v[0], acc_sh.at[idx_v[m8].at[0]], sem_s[msem],
                         add=True)

      def cnt_sc_wait(w, m8, msem, m4):
        pltpu.make_async_copy(g_v[0], acc_sh.at[idx_v[m8].at[0]],
                              sem_s[msem]).wait()

      run_pass(cnt_sc_issue, cnt_sc_wait, cnt_gather_issue, cnt_gather_wait,
               cntr.at[c], w0=c * CWIN, nw=CWIN)

    for j in range(fpc):
      fb = c * fpc + j
      gi, gw, si, sw = mk_gather(fb)
      run_pass(si, sw, gi, gw, segr.at[fb])

  return pl.kernel(body, out_type=tuple(out_type), mesh=mesh,
                   scratch_types=scratch)


def _spmm_first(*args):
  return _make_spmm(2, True)(*args)


def _spmm_mid(*args):
  return _make_spmm(2, False)(*args)


def _mm0_body(x_ref, w_ref, o_ref):
  o_ref[0] = jnp.dot(x_ref[...], w_ref[...],
                     preferred_element_type=jnp.float32)


def _mm0h(x, wh):
  return pl.pallas_call(
      _mm0_body,
      grid=(RB, 2),
      in_specs=[
          pl.BlockSpec((BR, 256), lambda r, n: (r, 0)),
          pl.BlockSpec((256, 128), lambda r, n: (0, n)),
      ],
      out_specs=pl.BlockSpec((1, BR, 128), lambda r, n: (n, r, 0)),
      out_shape=jax.ShapeDtypeStruct((2, N, 128), jnp.float32),
      compiler_params=pltpu.CompilerParams(
          dimension_semantics=("parallel", "parallel")),
  )(x, wh)


def _act(seg_ref, scl_ref, b_ref):
  t = (seg_ref[0] * (1.0 / (scl_ref[0, :, 0:1] + scl_ref[1, :, 0:1]))
       + b_ref[0, 0])
  return jnp.where(t >= 0, t, 0.2 * t)


def _midp_body(seg_ref, scl_ref, b_ref, w_ref, o_ref):
  k = pl.program_id(2)
  p = jnp.dot(_act(seg_ref, scl_ref, b_ref), w_ref[...],
              preferred_element_type=jnp.float32)

  @pl.when(k == 0)
  def _():
    o_ref[0] = p

  @pl.when(k > 0)
  def _():
    o_ref[0] += p


def _midp_acc_body(seg_ref, scl_ref, b_ref, w_ref, a_ref, o_ref):
  k = pl.program_id(2)
  p = jnp.dot(_act(seg_ref, scl_ref, b_ref), w_ref[...],
              preferred_element_type=jnp.float32)

  @pl.when(k == 0)
  def _():
    o_ref[0] = a_ref[0] + p

  @pl.when(k > 0)
  def _():
    o_ref[0] += p


def _midp(seg, scl, b2, wq, acc, n_out, noff=0):
  in_specs = [
      pl.BlockSpec((1, BR, 128), lambda r, n, k: (k, r, 0)),
      pl.BlockSpec((2, BR, 128), lambda r, n, k: (0, r, 0)),
      pl.BlockSpec((1, 1, 128), lambda r, n, k: (k, 0, 0)),
      pl.BlockSpec((128, 128), lambda r, n, k: (k, n)),
  ]
  args = [seg, scl, b2, wq]
  body = _midp_body
  if acc is not None:
    in_specs.append(
        pl.BlockSpec((1, BR, 128), lambda r, n, k, o=noff: (n + o, r, 0)))
    args.append(acc)
    body = _midp_acc_body
  return pl.pallas_call(
      body,
      grid=(RB, n_out, 2),
      in_specs=in_specs,
      out_specs=pl.BlockSpec((1, BR, 128), lambda r, n, k: (n, r, 0)),
      out_shape=jax.ShapeDtypeStruct((n_out, N, 128), jnp.float32),
      compiler_params=pltpu.CompilerParams(
          dimension_semantics=("parallel", "parallel", "arbitrary")),
  )(*args)


def _fin_body(seg_ref, scl_ref, b_ref, o_ref):
  sc = 1.0 / (scl_ref[0, :, 0:1] + scl_ref[1, :, 0:1])
  t0 = seg_ref[0] * sc + b_ref[0]
  t1 = seg_ref[1] * sc + b_ref[1]
  ss = jnp.sum(t0 * t0 + t1 * t1, axis=1, keepdims=True)
  inv = 1.0 / jnp.maximum(jnp.sqrt(ss), 1e-12)
  o_ref[:, :128] = t0 * inv
  o_ref[:, 128:] = t1 * inv


def _fin(seg, scl, b):
  return pl.pallas_call(
      _fin_body,
      grid=(RB,),
      in_specs=[
          pl.BlockSpec((2, BR, 128), lambda r: (0, r, 0)),
          pl.BlockSpec((2, BR, 128), lambda r: (0, r, 0)),
          pl.BlockSpec((2, 128), lambda r: (0, 0)),
      ],
      out_specs=pl.BlockSpec((BR, 256), lambda r: (r, 0)),
      out_shape=jax.ShapeDtypeStruct((N, 256), jnp.float32),
      compiler_params=pltpu.CompilerParams(
          dimension_semantics=("parallel",)),
  )(seg, scl, b)


def kernel(x, rows, cols, vals, w0, b0, w1, b1, w2, b2, w3, b3, w4, b4):
  e = rows.shape[0]
  pad = EP - e
  cols_p = jnp.concatenate([cols.astype(jnp.int32),
                            jnp.zeros((pad,), jnp.int32)])
  rows_p = jnp.concatenate([rows.astype(jnp.int32),
                            jnp.full((pad,), N, jnp.int32)])
  idx = jnp.stack([rows_p.reshape(-1, W), cols_p.reshape(-1, W)], axis=1)
  zeros = jnp.zeros((ZROWS, 128), jnp.float32)
  ones = jnp.ones((W, 128), jnp.float32)

  hw_a = _mm0h(x, w0[:, :256])
  hw_b = _mm0h(x, w0[:, 256:])
  seg_a, scl = _spmm_first(hw_a, idx, zeros, ones)
  (seg_b,) = _spmm_mid(hw_b, idx, zeros)
  ws = [w1, w2, w3]
  bs = [b0, b1, b2, b3]
  for i in range(3):
    b2a = bs[i].reshape(4, 1, 128)[0:2]
    b2b = bs[i].reshape(4, 1, 128)[2:4]
    wi = ws[i]
    p = _midp(seg_a, scl, b2a, wi[:256, :], None, 4)
    hw_a = _midp(seg_b, scl, b2b, wi[256:, :256], p, 2, noff=0)
    hw_b = _midp(seg_b, scl, b2b, wi[256:, 256:], p, 2, noff=2)
    (seg_a,) = _spmm_mid(hw_a, idx, zeros)
    (seg_b,) = _spmm_mid(hw_b, idx, zeros)
  b2a = b3.reshape(4, 1, 128)[0:2]
  b2b = b3.reshape(4, 1, 128)[2:4]
  p = _midp(seg_a, scl, b2a, w4[:256, :], None, 2)
  hw4_a = _midp(seg_b, scl, b2b, w4[256:, :128], p, 1, noff=0)
  hw4_b = _midp(seg_b, scl, b2b, w4[256:, 128:], p, 1, noff=1)
  hw4 = jnp.concatenate([hw4_a, hw4_b], axis=0)
  (seg,) = _spmm_mid(hw4, idx, zeros)
  return _fin(seg, scl, b4.reshape(2, 128))
